# baseline (speedup 1.0000x reference)
import functools
import math

import jax
import jax.numpy as jnp
from jax import lax
from jax.experimental import pallas as pl
from jax.experimental.pallas import tpu as pltpu
from jax.experimental.pallas import tpu_sc as plsc

_PI = math.pi
_NSLICES = 4
_BM = 2048


def _vq_idx(v, d0, n0, d1, n1):
    i0 = jnp.clip((v / d0).astype(jnp.int32), 0, n0 - 1)
    r = v - i0.astype(jnp.float32) * d0
    i1 = jnp.clip((r / d1).astype(jnp.int32), 0, n1 - 1)
    return i0, i1


def _sc_feat_body(ntok, chunk, goff, ns, xs, ys, hs, ms, t0, t1, h0, h1,
                  feat_hbm, xv, yv, hv, mv, t0v, t1v, h0v, h1v, fb0, fb1,
                  sem_in, sem0, sem1):
    wid = lax.axis_index("s") * 2 + lax.axis_index("c")
    nrow = ntok // 2
    gbase_a = goff + wid * nrow
    gbase_b = gbase_a + ns // 2

    copies = [
        pltpu.async_copy(xs.at[pl.ds(gbase_a, nrow)], xv.at[pl.ds(0, nrow)],
                         sem_in),
        pltpu.async_copy(xs.at[pl.ds(gbase_b, nrow)],
                         xv.at[pl.ds(nrow, nrow)], sem_in),
        pltpu.async_copy(ys.at[pl.ds(gbase_a, nrow)], yv.at[pl.ds(0, nrow)],
                         sem_in),
        pltpu.async_copy(ys.at[pl.ds(gbase_b, nrow)],
                         yv.at[pl.ds(nrow, nrow)], sem_in),
        pltpu.async_copy(hs.at[pl.ds(gbase_a, nrow)], hv.at[pl.ds(0, nrow)],
                         sem_in),
        pltpu.async_copy(hs.at[pl.ds(gbase_b, nrow)],
                         hv.at[pl.ds(nrow, nrow)], sem_in),
        pltpu.async_copy(ms.at[pl.ds(gbase_a, nrow)], mv.at[pl.ds(0, nrow)],
                         sem_in),
        pltpu.async_copy(ms.at[pl.ds(gbase_b, nrow)],
                         mv.at[pl.ds(nrow, nrow)], sem_in),
        pltpu.async_copy(t0, t0v, sem_in),
        pltpu.async_copy(t1, t1v, sem_in),
        pltpu.async_copy(h0, h0v, sem_in),
        pltpu.async_copy(h1, h1v, sem_in),
    ]
    for cp in copies:
        cp.wait()

    iota = lax.iota(jnp.int32, 16)
    ngroups = chunk // 16
    nchunks = nrow // chunk
    fbs = (fb0, fb1)
    sems = (sem0, sem1)
    descs = {}

    def half_feat(fb, rows, off, cbase):
        x = xv[pl.ds(off, 16)]
        y = yv[pl.ds(off, 16)]
        h = hv[pl.ds(off, 16)]
        m = mv[pl.ds(off, 16)]
        tx = x + 300.0
        ty = y + 300.0
        th = (h * 180.0) / _PI + 180.0
        ix0, ix1 = _vq_idx(tx, 1.0, 600, 0.01, 100)
        iy0, iy1 = _vq_idx(ty, 1.0, 600, 0.01, 100)
        ih0, ih1 = _vq_idx(th, 20.0, 20, 1.0, 20)
        gx0 = ix0 * 12
        gx1 = ix1 * 12
        gy0 = iy0 * 12
        gy1 = iy1 * 12
        gh0 = ih0 * 3
        gh1 = ih1 * 3
        col = lambda j: jnp.full((16,), cbase + j, jnp.int32)
        for j in range(12):
            plsc.store_scatter(fb, [rows, col(j)],
                               plsc.load_gather(t0v, [gx0 + j]))
            plsc.store_scatter(fb, [rows, col(12 + j)],
                               plsc.load_gather(t1v, [gx1 + j]))
            plsc.store_scatter(fb, [rows, col(24 + j)],
                               plsc.load_gather(t0v, [gy0 + j]))
            plsc.store_scatter(fb, [rows, col(36 + j)],
                               plsc.load_gather(t1v, [gy1 + j]))
        for j in range(3):
            plsc.store_scatter(fb, [rows, col(48 + j)],
                               plsc.load_gather(h0v, [gh0 + j]))
            plsc.store_scatter(fb, [rows, col(51 + j)],
                               plsc.load_gather(h1v, [gh1 + j]))
        plsc.store_scatter(fb, [rows, col(54)], m)

    for c in range(nchunks):
        b = c % 2
        if c >= 2:
            descs[b].wait()
        fb = fbs[b]

        @plsc.parallel_loop(0, ngroups, unroll=1)
        def _(g, fb=fb, c=c):
            off = c * chunk + g * 16
            rows = g * 16 + iota
            half_feat(fb, rows, off, 0)
            half_feat(fb, rows, nrow + off, 64)

        descs[b] = pltpu.async_copy(
            fb, feat_hbm.at[pl.ds(wid * nrow + c * chunk, chunk), :],
            sems[b])

    for b in range(min(nchunks, 2)):
        descs[b].wait()


def _sc_feat(xs, ys, hs, ms, t0p, t1p, h0p, h1p, ns, slice_off):
    nw = 32
    ntok = ns // nw
    nrow = ntok // 2
    chunk = min(128, nrow)
    mesh = plsc.VectorSubcoreMesh(core_axis_name="c", subcore_axis_name="s")
    return pl.kernel(
        functools.partial(_sc_feat_body, ntok, chunk, slice_off, ns),
        out_type=jax.ShapeDtypeStruct((ns // 2, 128), jnp.int32),
        mesh=mesh,
        compiler_params=pltpu.CompilerParams(needs_layout_passes=False),
        scratch_types=[
            pltpu.VMEM((ntok,), jnp.float32),
            pltpu.VMEM((ntok,), jnp.float32),
            pltpu.VMEM((ntok,), jnp.float32),
            pltpu.VMEM((ntok,), jnp.int32),
            pltpu.VMEM((600 * 12,), jnp.int32),
            pltpu.VMEM((100 * 12,), jnp.int32),
            pltpu.VMEM((20 * 3,), jnp.int32),
            pltpu.VMEM((20 * 3,), jnp.int32),
            pltpu.VMEM((chunk, 128), jnp.int32),
            pltpu.VMEM((chunk, 128), jnp.int32),
            pltpu.SemaphoreType.DMA,
            pltpu.SemaphoreType.DMA,
            pltpu.SemaphoreType.DMA,
        ],
    )(xs, ys, hs, ms, t0p, t1p, h0p, h1p)


def _pack_bf16_pairs(t):
    u = lax.bitcast_convert_type(t.astype(jnp.bfloat16), jnp.uint16)
    w = u[:, 0::2].astype(jnp.uint32) | (u[:, 1::2].astype(jnp.uint32) << 16)
    return lax.bitcast_convert_type(w, jnp.int32).reshape(-1)


def _ln(x, g, b, eps=1e-5):
    mu = jnp.mean(x, axis=-1, keepdims=True)
    xc = x - mu
    var = jnp.mean(xc * xc, axis=-1, keepdims=True)
    return xc * lax.rsqrt(var + eps) * g + b


def _mlp_compute(feat_ref, w0e_ref, w0o_ref, b0_ref, g0_ref,
                 be0_ref, w1_ref, b1_ref, g1_ref, be1_ref, w2_ref, b2_ref,
                 oob_ref, out_ref):
    wfull = feat_ref[...]
    for half in (0, 1):
        w = wfull[:, half * 64:half * 64 + 55]
        valid = w[:, 54:55] != 0
        packed = w[:, :54]
        fe = lax.bitcast_convert_type(
            lax.shift_left(packed, 16), jnp.float32).astype(jnp.bfloat16)
        fo = lax.bitcast_convert_type(
            lax.bitwise_and(packed, jnp.int32(-65536)),
            jnp.float32).astype(jnp.bfloat16)
        h = jnp.dot(fe, w0e_ref[...], preferred_element_type=jnp.float32)
        h = h + jnp.dot(fo, w0o_ref[...], preferred_element_type=jnp.float32)
        h = _ln(h + b0_ref[...], g0_ref[...], be0_ref[...])
        h = jnp.maximum(h, 0.0).astype(jnp.bfloat16)
        h = jnp.dot(h, w1_ref[...], preferred_element_type=jnp.float32)
        h = _ln(h + b1_ref[...], g1_ref[...], be1_ref[...])
        h = jnp.maximum(h, 0.0).astype(jnp.bfloat16)
        h = jnp.dot(h, w2_ref[...], preferred_element_type=jnp.float32)
        h = h + b2_ref[...]
        out_ref[half, :, :] = jnp.where(valid, h, oob_ref[...])


def _mlp_body_first(*refs):
    _mlp_compute(*refs)


def _mlp_body_chained(buf_ref, *refs):
    del buf_ref
    _mlp_compute(*refs)


def _mlp_slice(buf, feat2d, w0e, w0o, b0, g0, be0, w1, b1, g1, be1,
               w2, b2, oob, nslices, slice_idx, bm):
    ns2 = feat2d.shape[0]
    full = lambda shape: pl.BlockSpec(shape, lambda i: (0, 0))
    in_specs = [
        pl.BlockSpec((bm, 128), lambda i: (i, 0)),
        full((54, 256)), full((54, 256)),
        full((1, 256)), full((1, 256)), full((1, 256)),
        full((256, 256)),
        full((1, 256)), full((1, 256)), full((1, 256)),
        full((256, 256)),
        full((1, 256)), full((1, 256)),
    ]
    args = (feat2d, w0e, w0o, b0, g0, be0, w1, b1, g1, be1, w2, b2, oob)
    body = _mlp_body_first
    aliases = {}
    if buf is not None:
        in_specs = [pl.BlockSpec(memory_space=pl.ANY)] + in_specs
        args = (buf,) + args
        body = _mlp_body_chained
        aliases = {0: 0}
    s = slice_idx
    return pl.pallas_call(
        body,
        grid=(ns2 // bm,),
        in_specs=in_specs,
        out_specs=pl.BlockSpec((2, bm, 256), lambda i: (s, i, 0)),
        out_shape=jax.ShapeDtypeStruct((2 * nslices, ns2, 256), jnp.float32),
        input_output_aliases=aliases,
    )(*args)


def kernel(position, heading, valid_mask, pos_table0, pos_table1,
           head_table0, head_table1, W0, b0, g0, be0, W1, b1, g1, be1,
           W2, b2, oob_w):
    B, A, T = heading.shape
    n = B * A * T
    xs = position[..., 0].reshape(n)
    ys = position[..., 1].reshape(n)
    hs = heading.reshape(n)
    ms = valid_mask.astype(jnp.int32).reshape(n)

    t0p = _pack_bf16_pairs(pos_table0)
    t1p = _pack_bf16_pairs(pos_table1)
    h0p = _pack_bf16_pairs(head_table0)
    h1p = _pack_bf16_pairs(head_table1)

    row = lambda v: v.reshape(1, 256)
    w0b = W0.astype(jnp.bfloat16)
    weights = (w0b[0::2], w0b[1::2], row(b0), row(g0), row(be0),
               W1.astype(jnp.bfloat16), row(b1), row(g1), row(be1),
               W2.astype(jnp.bfloat16), row(b2), row(oob_w))

    ns = n // _NSLICES
    feats = [
        _sc_feat(xs, ys, hs, ms, t0p, t1p, h0p, h1p, ns, s * ns)
        for s in range(_NSLICES)
    ]
    buf = None
    for s in range(_NSLICES):
        buf = _mlp_slice(buf, feats[s], *weights, _NSLICES, s, _BM)
    return buf.reshape(B, A, T, 256)


# speedup vs baseline: 1.0400x; 1.0400x over previous
import functools
import math

import jax
import jax.numpy as jnp
from jax import lax
from jax.experimental import pallas as pl
from jax.experimental.pallas import tpu as pltpu
from jax.experimental.pallas import tpu_sc as plsc

_PI = math.pi
_NSLICES = 2
_BM = 2048


def _vq_idx(v, d0, n0, d1, n1):
    i0 = jnp.clip((v / d0).astype(jnp.int32), 0, n0 - 1)
    r = v - i0.astype(jnp.float32) * d0
    i1 = jnp.clip((r / d1).astype(jnp.int32), 0, n1 - 1)
    return i0, i1


def _sc_feat_body(ntok, chunk, goff, ns, xs, ys, hs, ms, t0, t1, h0, h1,
                  feat_hbm, xv, yv, hv, mv, t0v, t1v, h0v, h1v, fb0, fb1,
                  sem_in, sem0, sem1):
    wid = lax.axis_index("s") * 2 + lax.axis_index("c")
    nrow = ntok // 2
    gbase_a = goff + wid * nrow
    gbase_b = gbase_a + ns // 2

    copies = [
        pltpu.async_copy(xs.at[pl.ds(gbase_a, nrow)], xv.at[pl.ds(0, nrow)],
                         sem_in),
        pltpu.async_copy(xs.at[pl.ds(gbase_b, nrow)],
                         xv.at[pl.ds(nrow, nrow)], sem_in),
        pltpu.async_copy(ys.at[pl.ds(gbase_a, nrow)], yv.at[pl.ds(0, nrow)],
                         sem_in),
        pltpu.async_copy(ys.at[pl.ds(gbase_b, nrow)],
                         yv.at[pl.ds(nrow, nrow)], sem_in),
        pltpu.async_copy(hs.at[pl.ds(gbase_a, nrow)], hv.at[pl.ds(0, nrow)],
                         sem_in),
        pltpu.async_copy(hs.at[pl.ds(gbase_b, nrow)],
                         hv.at[pl.ds(nrow, nrow)], sem_in),
        pltpu.async_copy(ms.at[pl.ds(gbase_a, nrow)], mv.at[pl.ds(0, nrow)],
                         sem_in),
        pltpu.async_copy(ms.at[pl.ds(gbase_b, nrow)],
                         mv.at[pl.ds(nrow, nrow)], sem_in),
        pltpu.async_copy(t0, t0v, sem_in),
        pltpu.async_copy(t1, t1v, sem_in),
        pltpu.async_copy(h0, h0v, sem_in),
        pltpu.async_copy(h1, h1v, sem_in),
    ]
    for cp in copies:
        cp.wait()

    iota = lax.iota(jnp.int32, 16)
    ngroups = chunk // 16
    nchunks = nrow // chunk
    fbs = (fb0, fb1)
    sems = (sem0, sem1)
    descs = {}

    def half_feat(fb, rows, off, cbase):
        x = xv[pl.ds(off, 16)]
        y = yv[pl.ds(off, 16)]
        h = hv[pl.ds(off, 16)]
        m = mv[pl.ds(off, 16)]
        tx = x + 300.0
        ty = y + 300.0
        th = (h * 180.0) / _PI + 180.0
        ix0, ix1 = _vq_idx(tx, 1.0, 600, 0.01, 100)
        iy0, iy1 = _vq_idx(ty, 1.0, 600, 0.01, 100)
        ih0, ih1 = _vq_idx(th, 20.0, 20, 1.0, 20)
        gx0 = ix0 * 12
        gx1 = ix1 * 12
        gy0 = iy0 * 12
        gy1 = iy1 * 12
        gh0 = ih0 * 3
        gh1 = ih1 * 3
        col = lambda j: jnp.full((16,), cbase + j, jnp.int32)
        for j in range(12):
            plsc.store_scatter(fb, [rows, col(j)],
                               plsc.load_gather(t0v, [gx0 + j]))
            plsc.store_scatter(fb, [rows, col(12 + j)],
                               plsc.load_gather(t1v, [gx1 + j]))
            plsc.store_scatter(fb, [rows, col(24 + j)],
                               plsc.load_gather(t0v, [gy0 + j]))
            plsc.store_scatter(fb, [rows, col(36 + j)],
                               plsc.load_gather(t1v, [gy1 + j]))
        for j in range(3):
            plsc.store_scatter(fb, [rows, col(48 + j)],
                               plsc.load_gather(h0v, [gh0 + j]))
            plsc.store_scatter(fb, [rows, col(51 + j)],
                               plsc.load_gather(h1v, [gh1 + j]))
        plsc.store_scatter(fb, [rows, col(54)], m)

    for c in range(nchunks):
        b = c % 2
        if c >= 2:
            descs[b].wait()
        fb = fbs[b]

        @plsc.parallel_loop(0, ngroups, unroll=2)
        def _(g, fb=fb, c=c):
            off = c * chunk + g * 16
            rows = g * 16 + iota
            half_feat(fb, rows, off, 0)
            half_feat(fb, rows, nrow + off, 64)

        descs[b] = pltpu.async_copy(
            fb, feat_hbm.at[pl.ds(wid * nrow + c * chunk, chunk), :],
            sems[b])

    for b in range(min(nchunks, 2)):
        descs[b].wait()


def _sc_feat(xs, ys, hs, ms, t0p, t1p, h0p, h1p, ns, slice_off):
    nw = 32
    ntok = ns // nw
    nrow = ntok // 2
    chunk = min(128, nrow)
    mesh = plsc.VectorSubcoreMesh(core_axis_name="c", subcore_axis_name="s")
    return pl.kernel(
        functools.partial(_sc_feat_body, ntok, chunk, slice_off, ns),
        out_type=jax.ShapeDtypeStruct((ns // 2, 128), jnp.int32),
        mesh=mesh,
        compiler_params=pltpu.CompilerParams(needs_layout_passes=False),
        scratch_types=[
            pltpu.VMEM((ntok,), jnp.float32),
            pltpu.VMEM((ntok,), jnp.float32),
            pltpu.VMEM((ntok,), jnp.float32),
            pltpu.VMEM((ntok,), jnp.int32),
            pltpu.VMEM((600 * 12,), jnp.int32),
            pltpu.VMEM((100 * 12,), jnp.int32),
            pltpu.VMEM((20 * 3,), jnp.int32),
            pltpu.VMEM((20 * 3,), jnp.int32),
            pltpu.VMEM((chunk, 128), jnp.int32),
            pltpu.VMEM((chunk, 128), jnp.int32),
            pltpu.SemaphoreType.DMA,
            pltpu.SemaphoreType.DMA,
            pltpu.SemaphoreType.DMA,
        ],
    )(xs, ys, hs, ms, t0p, t1p, h0p, h1p)


def _pack_bf16_pairs(t):
    u = lax.bitcast_convert_type(t.astype(jnp.bfloat16), jnp.uint16)
    w = u[:, 0::2].astype(jnp.uint32) | (u[:, 1::2].astype(jnp.uint32) << 16)
    return lax.bitcast_convert_type(w, jnp.int32).reshape(-1)


def _ln(x, g, b, eps=1e-5):
    mu = jnp.mean(x, axis=-1, keepdims=True)
    xc = x - mu
    var = jnp.mean(xc * xc, axis=-1, keepdims=True)
    return xc * lax.rsqrt(var + eps) * g + b


def _mlp_compute(feat_ref, w0e_ref, w0o_ref, b0_ref, g0_ref,
                 be0_ref, w1_ref, b1_ref, g1_ref, be1_ref, w2_ref, b2_ref,
                 oob_ref, out_ref):
    wfull = feat_ref[...]
    for half in (0, 1):
        w = wfull[:, half * 64:half * 64 + 55]
        valid = w[:, 54:55] != 0
        packed = w[:, :54]
        fe = lax.bitcast_convert_type(
            lax.shift_left(packed, 16), jnp.float32).astype(jnp.bfloat16)
        fo = lax.bitcast_convert_type(
            lax.bitwise_and(packed, jnp.int32(-65536)),
            jnp.float32).astype(jnp.bfloat16)
        h = jnp.dot(fe, w0e_ref[...], preferred_element_type=jnp.float32)
        h = h + jnp.dot(fo, w0o_ref[...], preferred_element_type=jnp.float32)
        h = _ln(h + b0_ref[...], g0_ref[...], be0_ref[...])
        h = jnp.maximum(h, 0.0).astype(jnp.bfloat16)
        h = jnp.dot(h, w1_ref[...], preferred_element_type=jnp.float32)
        h = _ln(h + b1_ref[...], g1_ref[...], be1_ref[...])
        h = jnp.maximum(h, 0.0).astype(jnp.bfloat16)
        h = jnp.dot(h, w2_ref[...], preferred_element_type=jnp.float32)
        h = h + b2_ref[...]
        out_ref[half, :, :] = jnp.where(valid, h, oob_ref[...])


def _mlp_body_first(*refs):
    _mlp_compute(*refs)


def _mlp_body_chained(buf_ref, *refs):
    del buf_ref
    _mlp_compute(*refs)


def _mlp_slice(buf, feat2d, w0e, w0o, b0, g0, be0, w1, b1, g1, be1,
               w2, b2, oob, nslices, slice_idx, bm):
    ns2 = feat2d.shape[0]
    full = lambda shape: pl.BlockSpec(shape, lambda i: (0, 0))
    in_specs = [
        pl.BlockSpec((bm, 128), lambda i: (i, 0)),
        full((54, 256)), full((54, 256)),
        full((1, 256)), full((1, 256)), full((1, 256)),
        full((256, 256)),
        full((1, 256)), full((1, 256)), full((1, 256)),
        full((256, 256)),
        full((1, 256)), full((1, 256)),
    ]
    args = (feat2d, w0e, w0o, b0, g0, be0, w1, b1, g1, be1, w2, b2, oob)
    body = _mlp_body_first
    aliases = {}
    if buf is not None:
        in_specs = [pl.BlockSpec(memory_space=pl.ANY)] + in_specs
        args = (buf,) + args
        body = _mlp_body_chained
        aliases = {0: 0}
    s = slice_idx
    return pl.pallas_call(
        body,
        grid=(ns2 // bm,),
        in_specs=in_specs,
        out_specs=pl.BlockSpec((2, bm, 256), lambda i: (s, i, 0)),
        out_shape=jax.ShapeDtypeStruct((2 * nslices, ns2, 256), jnp.float32),
        input_output_aliases=aliases,
    )(*args)


def kernel(position, heading, valid_mask, pos_table0, pos_table1,
           head_table0, head_table1, W0, b0, g0, be0, W1, b1, g1, be1,
           W2, b2, oob_w):
    B, A, T = heading.shape
    n = B * A * T
    xs = position[..., 0].reshape(n)
    ys = position[..., 1].reshape(n)
    hs = heading.reshape(n)
    ms = valid_mask.astype(jnp.int32).reshape(n)

    t0p = _pack_bf16_pairs(pos_table0)
    t1p = _pack_bf16_pairs(pos_table1)
    h0p = _pack_bf16_pairs(head_table0)
    h1p = _pack_bf16_pairs(head_table1)

    row = lambda v: v.reshape(1, 256)
    w0b = W0.astype(jnp.bfloat16)
    weights = (w0b[0::2], w0b[1::2], row(b0), row(g0), row(be0),
               W1.astype(jnp.bfloat16), row(b1), row(g1), row(be1),
               W2.astype(jnp.bfloat16), row(b2), row(oob_w))

    ns = n // _NSLICES
    feats = [
        _sc_feat(xs, ys, hs, ms, t0p, t1p, h0p, h1p, ns, s * ns)
        for s in range(_NSLICES)
    ]
    buf = None
    for s in range(_NSLICES):
        buf = _mlp_slice(buf, feats[s], *weights, _NSLICES, s, _BM)
    return buf.reshape(B, A, T, 256)
